# baseline (speedup 1.0000x reference)
import functools

import jax
import jax.numpy as jnp
from jax import lax
from jax.experimental import pallas as pl
from jax.experimental.pallas import tpu as pltpu
from jax.experimental.pallas import tpu_sc as plsc

D = 64
TABLE_ROWS = 200
NC = 2
NS = 16
NW = NC * NS
IDX_BLK = 128
IDX_ROWS = 16
NB = 8
PF = 4
PAIR = 2 * IDX_ROWS


def _combine_body(v_ref, p_ref, o_ref):
    o_ref[...] = v_ref[...] + p_ref[...]


def _combine_tables(vocab_slice, pos_table):
    return pl.pallas_call(
        _combine_body,
        out_shape=jax.ShapeDtypeStruct((TABLE_ROWS, D), jnp.float32),
    )(vocab_slice, pos_table)


def _gather_body(nrows_pw, comb, x_hbm, out, idx_v, rows_v, gsem, ssem, isem):
    wid = lax.axis_index("s") * NC + lax.axis_index("c")
    base = wid * nrows_pw
    npairs = nrows_pw // PAIR

    def fire_gather(row, s, ib, j2):
        pltpu.async_copy(comb.at[idx_v.at[ib, j2]], rows_v.at[s], gsem.at[s])

    def wait_gather(s, ib, j2):
        pltpu.make_async_copy(
            comb.at[idx_v.at[ib, j2]], rows_v.at[s], gsem.at[s]
        ).wait()

    def fire_store(row, s):
        pltpu.async_copy(rows_v.at[s], out.at[pl.ds(row * IDX_BLK, IDX_BLK)],
                         ssem.at[s])

    def wait_store(row, s):
        pltpu.make_async_copy(rows_v.at[s],
                              out.at[pl.ds(row * IDX_BLK, IDX_BLK)],
                              ssem.at[s]).wait()

    def fire_idx(first_row, ib):
        pltpu.async_copy(x_hbm.at[pl.ds(first_row, IDX_ROWS)], idx_v.at[ib],
                         isem.at[ib])

    def wait_idx(first_row, ib):
        pltpu.make_async_copy(x_hbm.at[pl.ds(first_row, IDX_ROWS)],
                              idx_v.at[ib], isem.at[ib]).wait()

    def step(prow, p_has_next, j, first_pair, last_pair):
        s = j % NB
        ib, j2 = divmod(j, IDX_ROWS)
        row = prow + j
        wait_gather(s, ib, j2)
        fire_store(row, s)
        jp = (j + PF) % PAIR
        ibp, j2p = divmod(jp, IDX_ROWS)
        sp = (j + PF) % NB
        if not (last_pair and j >= PAIR - PF):
            if j == 12:
                wait_idx(prow + IDX_ROWS, 1)
            if j == 28:
                wait_idx(prow + PAIR, 0)
            if not (first_pair and j < PF):
                wait_store(row + PF - NB, sp)
            fire_gather(row + PF, sp, ibp, j2p)
        else:
            wait_store(row + PF - NB, sp)
        if j == IDX_ROWS - 1 and p_has_next:
            fire_idx(prow + PAIR, 0)
        if j == PAIR - 1 and p_has_next:
            fire_idx(prow + PAIR + IDX_ROWS, 1)

    pltpu.sync_copy(x_hbm.at[pl.ds(base, IDX_ROWS)], idx_v.at[0])
    fire_idx(base + IDX_ROWS, 1)
    for j in range(PF):
        fire_gather(base + j, j, 0, j)

    for j in range(PAIR):
        step(base, True, j, True, False)

    def pair_body(p, _):
        prow = base + p * PAIR
        for j in range(PAIR):
            step(prow, True, j, False, False)
        return 0

    lax.fori_loop(1, npairs - 1, pair_body, 0, unroll=False)

    lrow = base + (npairs - 1) * PAIR
    for j in range(PAIR):
        step(lrow, False, j, False, True)

    for j in range(PAIR - PF, PAIR):
        wait_store(lrow + j, j % NB)


def kernel(x, vocab_table, pos_table):
    B, S = x.shape
    total = B * S
    assert total % (NW * PAIR * IDX_BLK) == 0
    nrows = total // IDX_BLK
    nrows_pw = nrows // NW

    combined = _combine_tables(
        lax.slice(vocab_table, (0, 0), (TABLE_ROWS, D)), pos_table
    )

    x2 = x.reshape(nrows, IDX_BLK).astype(jnp.int32)

    mesh = plsc.VectorSubcoreMesh(core_axis_name="c", subcore_axis_name="s")
    out3 = pl.kernel(
        functools.partial(_gather_body, nrows_pw),
        out_type=jax.ShapeDtypeStruct((total, D), jnp.float32),
        mesh=mesh,
        scratch_types=[
            pltpu.VMEM((2, IDX_ROWS, IDX_BLK), jnp.int32),
            pltpu.VMEM((NB, IDX_BLK, D), jnp.float32),
            pltpu.SemaphoreType.DMA((NB,)),
            pltpu.SemaphoreType.DMA((NB,)),
            pltpu.SemaphoreType.DMA((2,)),
        ],
        compiler_params=pltpu.CompilerParams(use_tc_tiling_on_sc=False),
    )(combined, x2)

    return out3.reshape(B, S, D)


# speedup vs baseline: 1.5884x; 1.5884x over previous
import functools

import jax
import jax.numpy as jnp
from jax import lax
from jax.experimental import pallas as pl
from jax.experimental.pallas import tpu as pltpu
from jax.experimental.pallas import tpu_sc as plsc

D = 64
TABLE_ROWS = 200
NC = 2
NS = 16
NW = NC * NS
IDX_BLK = 128
IDX_ROWS = 16
NB = 8
PF = 4
PAIR = 2 * IDX_ROWS


def _combine_body(v_ref, p_ref, o_ref):
    o_ref[...] = jnp.broadcast_to((v_ref[...] + p_ref[...])[None], (NW, TABLE_ROWS, D))


def _combine_tables(vocab_slice, pos_table):
    return pl.pallas_call(
        _combine_body,
        out_shape=jax.ShapeDtypeStruct((NW, TABLE_ROWS, D), jnp.float32),
    )(vocab_slice, pos_table)


def _gather_body(nrows_pw, comb, x_hbm, out, idx_v, rows_v, gsem, ssem, isem):
    wid = lax.axis_index("s") * NC + lax.axis_index("c")
    base = wid * nrows_pw
    npairs = nrows_pw // PAIR
    myt = comb.at[wid]

    def fire_gather(row, s, ib, j2):
        pltpu.async_copy(myt.at[idx_v.at[ib, j2]], rows_v.at[s], gsem.at[s])

    def wait_gather(s, ib, j2):
        pltpu.make_async_copy(
            myt.at[idx_v.at[ib, j2]], rows_v.at[s], gsem.at[s]
        ).wait()

    def fire_store(row, s):
        pltpu.async_copy(rows_v.at[s], out.at[pl.ds(row * IDX_BLK, IDX_BLK)],
                         ssem.at[s])

    def wait_store(row, s):
        pltpu.make_async_copy(rows_v.at[s],
                              out.at[pl.ds(row * IDX_BLK, IDX_BLK)],
                              ssem.at[s]).wait()

    def fire_idx(first_row, ib):
        pltpu.async_copy(x_hbm.at[pl.ds(first_row, IDX_ROWS)], idx_v.at[ib],
                         isem.at[ib])

    def wait_idx(first_row, ib):
        pltpu.make_async_copy(x_hbm.at[pl.ds(first_row, IDX_ROWS)],
                              idx_v.at[ib], isem.at[ib]).wait()

    def step(prow, p_has_next, j, first_pair, last_pair):
        s = j % NB
        ib, j2 = divmod(j, IDX_ROWS)
        row = prow + j
        wait_gather(s, ib, j2)
        fire_store(row, s)
        jp = (j + PF) % PAIR
        ibp, j2p = divmod(jp, IDX_ROWS)
        sp = (j + PF) % NB
        if not (last_pair and j >= PAIR - PF):
            if j == 12:
                wait_idx(prow + IDX_ROWS, 1)
            if j == 28:
                wait_idx(prow + PAIR, 0)
            if not (first_pair and j < PF):
                wait_store(row + PF - NB, sp)
            fire_gather(row + PF, sp, ibp, j2p)
        else:
            wait_store(row + PF - NB, sp)
        if j == IDX_ROWS - 1 and p_has_next:
            fire_idx(prow + PAIR, 0)
        if j == PAIR - 1 and p_has_next:
            fire_idx(prow + PAIR + IDX_ROWS, 1)

    pltpu.sync_copy(x_hbm.at[pl.ds(base, IDX_ROWS)], idx_v.at[0])
    fire_idx(base + IDX_ROWS, 1)
    for j in range(PF):
        fire_gather(base + j, j, 0, j)

    for j in range(PAIR):
        step(base, True, j, True, False)

    def pair_body(p, _):
        prow = base + p * PAIR
        for j in range(PAIR):
            step(prow, True, j, False, False)
        return 0

    lax.fori_loop(1, npairs - 1, pair_body, 0, unroll=False)

    lrow = base + (npairs - 1) * PAIR
    for j in range(PAIR):
        step(lrow, False, j, False, True)

    for j in range(PAIR - PF, PAIR):
        wait_store(lrow + j, j % NB)


def kernel(x, vocab_table, pos_table):
    B, S = x.shape
    total = B * S
    assert total % (NW * PAIR * IDX_BLK) == 0
    nrows = total // IDX_BLK
    nrows_pw = nrows // NW

    combined = _combine_tables(
        lax.slice(vocab_table, (0, 0), (TABLE_ROWS, D)), pos_table
    )

    x2 = x.reshape(nrows, IDX_BLK).astype(jnp.int32)

    mesh = plsc.VectorSubcoreMesh(core_axis_name="c", subcore_axis_name="s")
    out3 = pl.kernel(
        functools.partial(_gather_body, nrows_pw),
        out_type=jax.ShapeDtypeStruct((total, D), jnp.float32),
        mesh=mesh,
        scratch_types=[
            pltpu.VMEM((2, IDX_ROWS, IDX_BLK), jnp.int32),
            pltpu.VMEM((NB, IDX_BLK, D), jnp.float32),
            pltpu.SemaphoreType.DMA((NB,)),
            pltpu.SemaphoreType.DMA((NB,)),
            pltpu.SemaphoreType.DMA((2,)),
        ],
        compiler_params=pltpu.CompilerParams(use_tc_tiling_on_sc=False),
    )(combined, x2)

    return out3.reshape(B, S, D)
